# baseline (speedup 1.0000x reference)
import jax
import jax.numpy as jnp
from jax import lax
from jax.experimental import pallas as pl
from jax.experimental.pallas import tpu as pltpu

_N = 20000
_K = 10
_ROWS = 160
_LANES = 128
_PAD_N = _ROWS * _LANES
_W = 1920.0
_H = 1080.0
_SCORE_THRESH = 0.3
_NMS_THRESH = 0.5
_TOPK = 100
_CLS_OFFSET = 4000.0
_BIG = 2 ** 30
_DEAD = -1082130432
_NEG = -(2 ** 31)


def _lex_better(v2, i2, v1, i1):
    return (v2 > v1) | ((v2 == v1) & (i2 < i1))


def _joint_argmax(vb, idx):
    v, i = vb, idx
    rows = _ROWS
    while rows > 1:
        half = rows // 2
        v1, i1 = v[:half], i[:half]
        v2, i2 = v[half:2 * half], i[half:2 * half]
        t = _lex_better(v2, i2, v1, i1)
        vn = jnp.where(t, v2, v1)
        in_ = jnp.where(t, i2, i1)
        if rows % 2:
            vl, il = v[2 * half:], i[2 * half:]
            t2 = _lex_better(vl[0:1], il[0:1], vn[0:1], in_[0:1])
            vn = jnp.where(jnp.logical_and(t2, lax.broadcasted_iota(
                jnp.int32, (half, _LANES), 0) == 0), vl, vn)
            in_ = jnp.where(jnp.logical_and(t2, lax.broadcasted_iota(
                jnp.int32, (half, _LANES), 0) == 0), il, in_)
        v, i = vn, in_
        rows = half
    lanes = _LANES
    while lanes > 1:
        half = lanes // 2
        v1, i1 = v[:, :half], i[:, :half]
        v2, i2 = v[:, half:lanes], i[:, half:lanes]
        t = _lex_better(v2, i2, v1, i1)
        v = jnp.where(t, v2, v1)
        i = jnp.where(t, i2, i1)
        lanes = half
    return v[0, 0], i[0, 0]


def _nms_kernel(sc_ref, bx_ref, out_ref, live, obx, areas, bxc):
    f32 = jnp.float32
    lane2d = lax.broadcasted_iota(jnp.int32, (_ROWS, _LANES), 1)
    row2d = lax.broadcasted_iota(jnp.int32, (_ROWS, _LANES), 0)
    flat = row2d * _LANES + lane2d
    lane1 = lax.broadcasted_iota(jnp.int32, (1, _LANES), 1)

    x1 = jnp.clip(bx_ref[0], 0.0, _W)
    y1 = jnp.clip(bx_ref[1], 0.0, _H)
    x2 = jnp.clip(bx_ref[2], 0.0, _W)
    y2 = jnp.clip(bx_ref[3], 0.0, _H)
    coords = (x1, y1, x2, y2)
    for k in range(4):
        bxc[k] = coords[k]
    m0 = []
    i0 = []
    for c in range(_K):
        off = f32((c + 1) * _CLS_OFFSET)
        ox1 = x1 + off
        oy1 = y1 + off
        ox2 = x2 + off
        oy2 = y2 + off
        obx[4 * c + 0] = ox1
        obx[4 * c + 1] = oy1
        obx[4 * c + 2] = ox2
        obx[4 * c + 3] = oy2
        areas[c] = jnp.maximum(ox2 - ox1, 0.0) * jnp.maximum(oy2 - oy1, 0.0)
        s = sc_ref[c]
        vb = jnp.where(s > _SCORE_THRESH,
                       lax.bitcast_convert_type(s, jnp.int32), _DEAD)
        live[c] = vb
        mv, mi = _joint_argmax(vb, flat)
        m0.append(mv)
        i0.append(mi)

    bv = m0[0]
    bb = i0[0]
    bkey = i0[0] * _K
    bc = jnp.int32(0)
    for c in range(1, _K):
        key = i0[c] * _K + c
        t = _lex_better(m0[c], key, bv, bkey)
        bv = jnp.where(t, m0[c], bv)
        bb = jnp.where(t, i0[c], bb)
        bkey = jnp.where(t, key, bkey)
        bc = jnp.where(t, jnp.int32(c), bc)

    def step(t_idx, carry):
        m, i_, bc, bb, bv, bkey = carry
        alive = bv > 0
        r = bb // _LANES
        l = bb % _LANES
        sel = lane1 == l
        sx1 = jnp.max(jnp.where(sel, bxc[0, pl.ds(r, 1), :], -1.0))
        sy1 = jnp.max(jnp.where(sel, bxc[1, pl.ds(r, 1), :], -1.0))
        sx2 = jnp.max(jnp.where(sel, bxc[2, pl.ds(r, 1), :], -1.0))
        sy2 = jnp.max(jnp.where(sel, bxc[3, pl.ds(r, 1), :], -1.0))
        off = (bc + 1).astype(f32) * _CLS_OFFSET
        px1 = sx1 + off
        py1 = sy1 + off
        px2 = sx2 + off
        py2 = sy2 + off
        a1 = jnp.maximum(px2 - px1, 0.0) * jnp.maximum(py2 - py1, 0.0)

        c4 = bc * 4
        ix1 = jnp.maximum(px1, obx[c4 + 0])
        iy1 = jnp.maximum(py1, obx[c4 + 1])
        ix2 = jnp.minimum(px2, obx[c4 + 2])
        iy2 = jnp.minimum(py2, obx[c4 + 3])
        inter = jnp.maximum(ix2 - ix1, 0.0) * jnp.maximum(iy2 - iy1, 0.0)
        iou = inter / (a1 + areas[bc] - inter + 1e-9)
        suppv = (iou > _NMS_THRESH) & alive
        vb_col = live[bc]
        newvb = jnp.where(suppv | (flat == bb), _DEAD, vb_col)
        live[bc] = newvb
        mx_v, mx_i = _joint_argmax(newvb, flat)

        ov = jnp.int32(_NEG)
        ob = jnp.int32(0)
        okey = jnp.int32(_BIG)
        oc = jnp.int32(0)
        for c in range(_K):
            mc = jnp.where(bc == c, jnp.int32(_NEG), m[c])
            key = i_[c] * _K + c
            t = _lex_better(mc, key, ov, okey)
            ov = jnp.where(t, mc, ov)
            ob = jnp.where(t, i_[c], ob)
            okey = jnp.where(t, key, okey)
            oc = jnp.where(t, jnp.int32(c), oc)

        score_f = lax.bitcast_convert_type(bv, f32)
        vals = (
            jnp.where(alive, sx1, 0.0),
            jnp.where(alive, sy1, 0.0),
            jnp.where(alive, sx2, 0.0),
            jnp.where(alive, sy2, 0.0),
            jnp.where(alive, score_f, 0.0),
            jnp.where(alive, (bc + 1).astype(f32), 0.0),
        )
        rowv = jnp.zeros((1, _LANES), f32)
        for k, v in enumerate(vals):
            rowv = jnp.where(lane1 == k, v, rowv)
        out_ref[pl.ds(t_idx, 1), :] = rowv

        mkey = mx_i * _K + bc
        t = _lex_better(mx_v, mkey, ov, okey)
        nbv = jnp.where(t, mx_v, ov)
        nbb = jnp.where(t, mx_i, ob)
        nbkey = jnp.where(t, mkey, okey)
        nbc = jnp.where(t, bc, oc)

        newm = tuple(jnp.where(bc == c, mx_v, m[c]) for c in range(_K))
        newi = tuple(jnp.where(bc == c, mx_i, i_[c]) for c in range(_K))
        return (newm, newi, nbc, nbb, nbv, nbkey)

    carry0 = (tuple(m0), tuple(i0), bc, bb, bv, bkey)
    lax.fori_loop(0, _TOPK, step, carry0, unroll=False)


@jax.jit
def kernel(boxes, scores):
    st = scores[:, 1:].T
    st = jnp.pad(st, ((0, 0), (0, _PAD_N - _N)))
    st = st.reshape(_K, _ROWS, _LANES)
    bt = boxes.T
    bt = jnp.pad(bt, ((0, 0), (0, _PAD_N - _N)))
    bt = bt.reshape(4, _ROWS, _LANES)

    out = pl.pallas_call(
        _nms_kernel,
        out_shape=jax.ShapeDtypeStruct((104, _LANES), jnp.float32),
        scratch_shapes=[
            pltpu.VMEM((_K, _ROWS, _LANES), jnp.int32),
            pltpu.VMEM((4 * _K, _ROWS, _LANES), jnp.float32),
            pltpu.VMEM((_K, _ROWS, _LANES), jnp.float32),
            pltpu.VMEM((4, _ROWS, _LANES), jnp.float32),
        ],
    )(st, bt)

    out_boxes = out[:_TOPK, 0:4]
    out_scores = out[:_TOPK, 4]
    out_cls = out[:_TOPK, 5].astype(jnp.int32)
    return out_boxes, out_scores, out_cls


# speedup vs baseline: 1.4748x; 1.4748x over previous
import jax
import jax.numpy as jnp
from jax import lax
from jax.experimental import pallas as pl
from jax.experimental.pallas import tpu as pltpu

_N = 20000
_K = 10
_ROWS = 160
_LANES = 128
_PAD_N = _ROWS * _LANES
_W = 1920.0
_H = 1080.0
_SCORE_THRESH = 0.3
_NMS_THRESH = 0.5
_TOPK = 100
_CLS_OFFSET = 4000.0
_BIG = 2 ** 30
_DEAD = -1082130432
_NEG = -(2 ** 31)


def _lex_better(v2, i2, v1, i1):
    return (v2 > v1) | ((v2 == v1) & (i2 < i1))


def _joint_argmax(lv, idx):
    mx = jnp.max(lv)
    mi = jnp.min(jnp.where(lv == mx, idx, _BIG))
    return lax.bitcast_convert_type(mx, jnp.int32), mi


def _nms_kernel(sc_ref, bx_ref, out_ref, live, obx, areas, bxc):
    f32 = jnp.float32
    lane2d = lax.broadcasted_iota(jnp.int32, (_ROWS, _LANES), 1)
    row2d = lax.broadcasted_iota(jnp.int32, (_ROWS, _LANES), 0)
    flat = row2d * _LANES + lane2d
    lane1 = lax.broadcasted_iota(jnp.int32, (1, _LANES), 1)

    x1 = jnp.clip(bx_ref[0], 0.0, _W)
    y1 = jnp.clip(bx_ref[1], 0.0, _H)
    x2 = jnp.clip(bx_ref[2], 0.0, _W)
    y2 = jnp.clip(bx_ref[3], 0.0, _H)
    coords = (x1, y1, x2, y2)
    for k in range(4):
        bxc[k] = coords[k]
    m0 = []
    i0 = []
    for c in range(_K):
        off = f32((c + 1) * _CLS_OFFSET)
        ox1 = x1 + off
        oy1 = y1 + off
        ox2 = x2 + off
        oy2 = y2 + off
        obx[4 * c + 0] = ox1
        obx[4 * c + 1] = oy1
        obx[4 * c + 2] = ox2
        obx[4 * c + 3] = oy2
        areas[c] = jnp.maximum(ox2 - ox1, 0.0) * jnp.maximum(oy2 - oy1, 0.0)
        s = sc_ref[c]
        lv = jnp.where(s > _SCORE_THRESH, s, -1.0)
        live[c] = lv
        mv, mi = _joint_argmax(lv, flat)
        m0.append(mv)
        i0.append(mi)

    bv = m0[0]
    bb = i0[0]
    bkey = i0[0] * _K
    bc = jnp.int32(0)
    for c in range(1, _K):
        key = i0[c] * _K + c
        t = _lex_better(m0[c], key, bv, bkey)
        bv = jnp.where(t, m0[c], bv)
        bb = jnp.where(t, i0[c], bb)
        bkey = jnp.where(t, key, bkey)
        bc = jnp.where(t, jnp.int32(c), bc)

    def step(t_idx, carry):
        m, i_, bc, bb, bv, bkey = carry
        alive = bv > 0
        r = bb // _LANES
        l = bb % _LANES
        sel = lane1 == l
        sx1 = jnp.max(jnp.where(sel, bxc[0, pl.ds(r, 1), :], -1.0))
        sy1 = jnp.max(jnp.where(sel, bxc[1, pl.ds(r, 1), :], -1.0))
        sx2 = jnp.max(jnp.where(sel, bxc[2, pl.ds(r, 1), :], -1.0))
        sy2 = jnp.max(jnp.where(sel, bxc[3, pl.ds(r, 1), :], -1.0))
        off = (bc + 1).astype(f32) * _CLS_OFFSET
        px1 = sx1 + off
        py1 = sy1 + off
        px2 = sx2 + off
        py2 = sy2 + off
        a1 = jnp.maximum(px2 - px1, 0.0) * jnp.maximum(py2 - py1, 0.0)

        c4 = bc * 4
        ix1 = jnp.maximum(px1, obx[c4 + 0])
        iy1 = jnp.maximum(py1, obx[c4 + 1])
        ix2 = jnp.minimum(px2, obx[c4 + 2])
        iy2 = jnp.minimum(py2, obx[c4 + 3])
        inter = jnp.maximum(ix2 - ix1, 0.0) * jnp.maximum(iy2 - iy1, 0.0)
        iou = inter / (a1 + areas[bc] - inter + 1e-9)
        suppv = (iou > _NMS_THRESH) & alive
        lv_col = live[bc]
        newlv = jnp.where(suppv | (flat == bb), -1.0, lv_col)
        live[bc] = newlv
        mx_v, mx_i = _joint_argmax(newlv, flat)

        ov = jnp.int32(_NEG)
        ob = jnp.int32(0)
        okey = jnp.int32(_BIG)
        oc = jnp.int32(0)
        for c in range(_K):
            mc = jnp.where(bc == c, jnp.int32(_NEG), m[c])
            key = i_[c] * _K + c
            t = _lex_better(mc, key, ov, okey)
            ov = jnp.where(t, mc, ov)
            ob = jnp.where(t, i_[c], ob)
            okey = jnp.where(t, key, okey)
            oc = jnp.where(t, jnp.int32(c), oc)

        score_f = lax.bitcast_convert_type(bv, f32)
        vals = (
            jnp.where(alive, sx1, 0.0),
            jnp.where(alive, sy1, 0.0),
            jnp.where(alive, sx2, 0.0),
            jnp.where(alive, sy2, 0.0),
            jnp.where(alive, score_f, 0.0),
            jnp.where(alive, (bc + 1).astype(f32), 0.0),
        )
        rowv = jnp.zeros((1, _LANES), f32)
        for k, v in enumerate(vals):
            rowv = jnp.where(lane1 == k, v, rowv)
        out_ref[pl.ds(t_idx, 1), :] = rowv

        mkey = mx_i * _K + bc
        t = _lex_better(mx_v, mkey, ov, okey)
        nbv = jnp.where(t, mx_v, ov)
        nbb = jnp.where(t, mx_i, ob)
        nbkey = jnp.where(t, mkey, okey)
        nbc = jnp.where(t, bc, oc)

        newm = tuple(jnp.where(bc == c, mx_v, m[c]) for c in range(_K))
        newi = tuple(jnp.where(bc == c, mx_i, i_[c]) for c in range(_K))
        return (newm, newi, nbc, nbb, nbv, nbkey)

    carry0 = (tuple(m0), tuple(i0), bc, bb, bv, bkey)
    lax.fori_loop(0, _TOPK, step, carry0, unroll=False)


@jax.jit
def kernel(boxes, scores):
    st = scores[:, 1:].T
    st = jnp.pad(st, ((0, 0), (0, _PAD_N - _N)))
    st = st.reshape(_K, _ROWS, _LANES)
    bt = boxes.T
    bt = jnp.pad(bt, ((0, 0), (0, _PAD_N - _N)))
    bt = bt.reshape(4, _ROWS, _LANES)

    out = pl.pallas_call(
        _nms_kernel,
        out_shape=jax.ShapeDtypeStruct((104, _LANES), jnp.float32),
        scratch_shapes=[
            pltpu.VMEM((_K, _ROWS, _LANES), jnp.float32),
            pltpu.VMEM((4 * _K, _ROWS, _LANES), jnp.float32),
            pltpu.VMEM((_K, _ROWS, _LANES), jnp.float32),
            pltpu.VMEM((4, _ROWS, _LANES), jnp.float32),
        ],
    )(st, bt)

    out_boxes = out[:_TOPK, 0:4]
    out_scores = out[:_TOPK, 4]
    out_cls = out[:_TOPK, 5].astype(jnp.int32)
    return out_boxes, out_scores, out_cls
